# baseline (speedup 1.0000x reference)
import functools

import jax
import jax.numpy as jnp
from jax import lax
from jax.experimental import pallas as pl
from jax.experimental.pallas import tpu as pltpu
from jax.experimental.pallas import tpu_sc as plsc

_VOCAB = 1000000
_DIM = 64
_BATCH = 16384
_LANES = 16

_NC = 2
_NS = 16
_NW = _NC * _NS

_SC_ROWS = 8192
_BPW = _SC_ROWS // _NW
_CHUNK = 32
_NCHUNK = _BPW // _CHUNK
_IROWS = _BPW // 128
_SROWS = _SC_ROWS // 128
_NBUF = 8
_NSEM = 4

_TC_ROWS = _BATCH - _SC_ROWS
_BLK = 1024
_W = 128
_NWAVE = _BLK // _W
_TC_NBUF = 2


def _lane_permute(x, idx):
    return lax.gather(
        x, idx[:, None],
        lax.GatherDimensionNumbers(
            offset_dims=(), collapsed_slice_dims=(0,), start_index_map=(0,)),
        slice_sizes=(1,),
        mode=lax.GatherScatterMode.PROMISE_IN_BOUNDS)


def _sc_body(t_hbm, c_hbm, tt_hbm, ct_hbm, out_hbm,
             tidx_v, cidx_v, trows, crows, scores, *sems):
    wid = lax.axis_index("s") * _NC + lax.axis_index("c")
    base = wid * _IROWS

    pltpu.sync_copy(t_hbm.at[pl.ds(base, _IROWS)], tidx_v)
    pltpu.sync_copy(c_hbm.at[pl.ds(base, _IROWS)], cidx_v)

    def chunk_scalars(ci):
        r, c0 = ci >> 2, (ci & (128 // _CHUNK - 1)) * _CHUNK
        tis, cis = [], []
        for v in range(_CHUNK // _LANES):
            tvec = tidx_v[r, pl.ds(c0 + v * _LANES, _LANES)]
            cvec = cidx_v[r, pl.ds(c0 + v * _LANES, _LANES)]
            tis += [tvec[l] for l in range(_LANES)]
            cis += [cvec[l] for l in range(_LANES)]
        return tis, cis

    def row_copies(ti, ci_, slot, j):
        t_cp = (tt_hbm.at[pl.ds(ti, 1)], trows.at[slot, pl.ds(j, 1)])
        c_cp = (ct_hbm.at[pl.ds(ci_, 1)], crows.at[slot, pl.ds(j, 1)])
        return t_cp, c_cp

    def fire(ci, slot):
        tis, cis = chunk_scalars(ci)
        for j in range(_CHUNK):
            t_cp, c_cp = row_copies(tis[j], cis[j], slot, j)
            pltpu.async_copy(*t_cp, sems[j % _NSEM])
            pltpu.async_copy(*c_cp, sems[j % _NSEM])

    def drain(ci, slot):
        tis, cis = chunk_scalars(ci)
        for j in range(_CHUNK):
            t_cp, c_cp = row_copies(tis[j], cis[j], slot, j)
            pltpu.make_async_copy(*t_cp, sems[j % _NSEM]).wait()
            pltpu.make_async_copy(*c_cp, sems[j % _NSEM]).wait()

    lane = lax.iota(jnp.int32, _LANES)
    stages = [(lane ^ h, (lane & h) == 0) for h in (8, 4, 2, 1)]
    bitrev = (((lane & 1) << 3) | ((lane & 2) << 1)
              | ((lane & 4) >> 1) | ((lane & 8) >> 3))

    def merge(a, b, perm_h, mask_h):
        u = a + _lane_permute(a, perm_h)
        v = b + _lane_permute(b, perm_h)
        return jnp.where(mask_h, u, v)

    def tree(vecs):
        for perm_h, mask_h in stages:
            vecs = [merge(vecs[i], vecs[i + 1], perm_h, mask_h)
                    for i in range(0, len(vecs), 2)]
        return _lane_permute(vecs[0], bitrev)

    for ci in range(_NBUF - 1):
        fire(ci, ci)

    def chunk_body(ci, _):
        slot = lax.rem(ci, _NBUF)
        drain(ci, slot)

        @pl.when(ci + _NBUF - 1 < _NCHUNK)
        def _():
            nxt = ci + _NBUF - 1
            fire(nxt, lax.rem(nxt, _NBUF))

        for g in range(_CHUNK // _LANES):
            vecs = []
            for r in range(_LANES):
                j = g * _LANES + r
                acc = (trows[slot, j, pl.ds(0, _LANES)]
                       * crows[slot, j, pl.ds(0, _LANES)])
                for k in range(1, _DIM // _LANES):
                    acc = acc + (trows[slot, j, pl.ds(k * _LANES, _LANES)]
                                 * crows[slot, j, pl.ds(k * _LANES, _LANES)])
                vecs.append(acc)
            totals = tree(vecs)
            flat = ci * _CHUNK + g * _LANES
            scores[flat >> 7, pl.ds(flat & 127, _LANES)] = totals
        return 0

    lax.fori_loop(0, _NCHUNK, chunk_body, 0)

    pltpu.sync_copy(scores, out_hbm.at[pl.ds(base, _IROWS)])


def _tc_body(tidx_ref, cidx_ref, tt_hbm, ct_hbm, out_ref,
             tbuf, cbuf, sem):
    i = pl.program_id(0)
    base = i * _BLK

    def wave_copies(w, slot):
        cps = []
        for jj in range(_W):
            row = base + w * _W + jj
            cps.append((tt_hbm.at[pl.ds(tidx_ref[row], 1)],
                        tbuf.at[slot, pl.ds(jj, 1)]))
            cps.append((ct_hbm.at[pl.ds(cidx_ref[row], 1)],
                        cbuf.at[slot, pl.ds(jj, 1)]))
        return cps

    def fire(w, slot):
        for cp in wave_copies(w, slot):
            pltpu.make_async_copy(*cp, sem).start()

    def drain(w, slot):
        for cp in wave_copies(w, slot):
            pltpu.make_async_copy(*cp, sem).wait()

    fire(0, 0)
    for w in range(_NWAVE):
        slot = w % _TC_NBUF
        drain(w, slot)
        if w + 1 < _NWAVE:
            fire(w + 1, (w + 1) % _TC_NBUF)
        prod = tbuf[slot] * cbuf[slot]
        out_ref[pl.ds(w * _W, _W)] = jnp.sum(prod, axis=1)


@jax.jit
def _forward(t_idx_sc, c_idx_sc, t_idx_tc, c_idx_tc,
             target_table, context_table):
    mesh = plsc.VectorSubcoreMesh(core_axis_name="c", subcore_axis_name="s")
    sc = functools.partial(
        pl.kernel,
        mesh=mesh,
        cost_estimate=pl.CostEstimate(
            flops=2 * _SC_ROWS * _DIM,
            bytes_accessed=2 * _SC_ROWS * _DIM * 4,
            transcendentals=0),
        out_type=jax.ShapeDtypeStruct((_SROWS, 128), jnp.float32),
        scratch_types=[
            pltpu.VMEM((_IROWS, 128), jnp.int32),
            pltpu.VMEM((_IROWS, 128), jnp.int32),
            pltpu.VMEM((_NBUF, _CHUNK, _DIM), jnp.float32),
            pltpu.VMEM((_NBUF, _CHUNK, _DIM), jnp.float32),
            pltpu.VMEM((_IROWS, 128), jnp.float32),
        ] + [pltpu.SemaphoreType.DMA] * _NSEM,
    )(_sc_body)
    out_sc = sc(t_idx_sc, c_idx_sc, target_table, context_table)

    grid_spec = pltpu.PrefetchScalarGridSpec(
        num_scalar_prefetch=2,
        grid=(_TC_ROWS // _BLK,),
        in_specs=[
            pl.BlockSpec(memory_space=pl.ANY),
            pl.BlockSpec(memory_space=pl.ANY),
        ],
        out_specs=pl.BlockSpec((_BLK,), lambda i, t, c: (i,)),
        scratch_shapes=[
            pltpu.VMEM((_TC_NBUF, _W, _DIM), jnp.float32),
            pltpu.VMEM((_TC_NBUF, _W, _DIM), jnp.float32),
            pltpu.SemaphoreType.DMA,
        ],
    )
    out_tc = pl.pallas_call(
        _tc_body,
        grid_spec=grid_spec,
        out_shape=jax.ShapeDtypeStruct((_TC_ROWS,), jnp.float32),
    )(t_idx_tc, c_idx_tc, target_table, context_table)

    return jnp.concatenate([out_sc.reshape(_SC_ROWS), out_tc])


def kernel(target, context, target_table, context_table):
    t_idx = target.astype(jnp.int32)
    c_idx = context.astype(jnp.int32)
    t_sc = t_idx[:_SC_ROWS].reshape(_SROWS, 128)
    c_sc = c_idx[:_SC_ROWS].reshape(_SROWS, 128)
    return _forward(t_sc, c_sc, t_idx[_SC_ROWS:], c_idx[_SC_ROWS:],
                    target_table, context_table)


# speedup vs baseline: 1.1445x; 1.1445x over previous
import functools

import jax
import jax.numpy as jnp
from jax import lax
from jax.experimental import pallas as pl
from jax.experimental.pallas import tpu as pltpu
from jax.experimental.pallas import tpu_sc as plsc

_VOCAB = 1000000
_DIM = 64
_BATCH = 16384
_LANES = 16

_NC = 2
_NS = 16
_NW = _NC * _NS
_BPW = _BATCH // _NW
_CHUNK = 32
_NCHUNK = _BPW // _CHUNK
_IROWS = _BPW // 128
_ROWS = _BATCH // 128
_NBUF = 8
_NSEM = 4


def _lane_permute(x, idx):
    return lax.gather(
        x, idx[:, None],
        lax.GatherDimensionNumbers(
            offset_dims=(), collapsed_slice_dims=(0,), start_index_map=(0,)),
        slice_sizes=(1,),
        mode=lax.GatherScatterMode.PROMISE_IN_BOUNDS)


def _sc_body(t_hbm, c_hbm, tt_hbm, ct_hbm, out_hbm,
             tidx_v, cidx_v, trows, crows, scores, *sems):
    wid = lax.axis_index("s") * _NC + lax.axis_index("c")
    base = wid * _IROWS

    pltpu.sync_copy(t_hbm.at[pl.ds(base, _IROWS)], tidx_v)
    pltpu.sync_copy(c_hbm.at[pl.ds(base, _IROWS)], cidx_v)

    def chunk_scalars(ci):
        r, c0 = ci >> 2, (ci & (128 // _CHUNK - 1)) * _CHUNK
        tis, cis = [], []
        for v in range(_CHUNK // _LANES):
            tvec = tidx_v[r, pl.ds(c0 + v * _LANES, _LANES)]
            cvec = cidx_v[r, pl.ds(c0 + v * _LANES, _LANES)]
            tis += [tvec[l] for l in range(_LANES)]
            cis += [cvec[l] for l in range(_LANES)]
        return tis, cis

    def row_copies(ti, ci_, slot, j):
        t_cp = (tt_hbm.at[pl.ds(ti, 1)], trows.at[slot, pl.ds(j, 1)])
        c_cp = (ct_hbm.at[pl.ds(ci_, 1)], crows.at[slot, pl.ds(j, 1)])
        return t_cp, c_cp

    def fire(ci, slot):
        tis, cis = chunk_scalars(ci)
        for j in range(_CHUNK):
            t_cp, c_cp = row_copies(tis[j], cis[j], slot, j)
            pltpu.async_copy(*t_cp, sems[j % _NSEM])
            pltpu.async_copy(*c_cp, sems[j % _NSEM])

    def drain(ci, slot):
        tis, cis = chunk_scalars(ci)
        for j in range(_CHUNK):
            t_cp, c_cp = row_copies(tis[j], cis[j], slot, j)
            pltpu.make_async_copy(*t_cp, sems[j % _NSEM]).wait()
            pltpu.make_async_copy(*c_cp, sems[j % _NSEM]).wait()

    lane = lax.iota(jnp.int32, _LANES)
    stages = [(lane ^ h, (lane & h) == 0) for h in (8, 4, 2, 1)]
    bitrev = (((lane & 1) << 3) | ((lane & 2) << 1)
              | ((lane & 4) >> 1) | ((lane & 8) >> 3))

    def merge(a, b, perm_h, mask_h):
        u = a + _lane_permute(a, perm_h)
        v = b + _lane_permute(b, perm_h)
        return jnp.where(mask_h, u, v)

    def tree(vecs):
        for perm_h, mask_h in stages:
            vecs = [merge(vecs[i], vecs[i + 1], perm_h, mask_h)
                    for i in range(0, len(vecs), 2)]
        return _lane_permute(vecs[0], bitrev)

    for ci in range(_NBUF - 1):
        fire(ci, ci)

    def chunk_body(ci, _):
        slot = lax.rem(ci, _NBUF)
        drain(ci, slot)

        @pl.when(ci + _NBUF - 1 < _NCHUNK)
        def _():
            nxt = ci + _NBUF - 1
            fire(nxt, lax.rem(nxt, _NBUF))

        for g in range(_CHUNK // _LANES):
            vecs = []
            for r in range(_LANES):
                j = g * _LANES + r
                acc = (trows[slot, j, pl.ds(0, _LANES)]
                       * crows[slot, j, pl.ds(0, _LANES)])
                for k in range(1, _DIM // _LANES):
                    acc = acc + (trows[slot, j, pl.ds(k * _LANES, _LANES)]
                                 * crows[slot, j, pl.ds(k * _LANES, _LANES)])
                vecs.append(acc)
            totals = tree(vecs)
            flat = ci * _CHUNK + g * _LANES
            scores[flat >> 7, pl.ds(flat & 127, _LANES)] = totals
        return 0

    lax.fori_loop(0, _NCHUNK, chunk_body, 0)

    pltpu.sync_copy(scores, out_hbm.at[pl.ds(base, _IROWS)])


@jax.jit
def _sc_scores(t_idx, c_idx, target_table, context_table):
    mesh = plsc.VectorSubcoreMesh(core_axis_name="c", subcore_axis_name="s")
    k = functools.partial(
        pl.kernel,
        mesh=mesh,
        out_type=jax.ShapeDtypeStruct((_ROWS, 128), jnp.float32),
        scratch_types=[
            pltpu.VMEM((_IROWS, 128), jnp.int32),
            pltpu.VMEM((_IROWS, 128), jnp.int32),
            pltpu.VMEM((_NBUF, _CHUNK, _DIM), jnp.float32),
            pltpu.VMEM((_NBUF, _CHUNK, _DIM), jnp.float32),
            pltpu.VMEM((_IROWS, 128), jnp.float32),
        ] + [pltpu.SemaphoreType.DMA] * _NSEM,
    )(_sc_body)
    return k(t_idx, c_idx, target_table, context_table)


def kernel(target, context, target_table, context_table):
    t_idx = target.astype(jnp.int32).reshape(_ROWS, 128)
    c_idx = context.astype(jnp.int32).reshape(_ROWS, 128)
    out = _sc_scores(t_idx, c_idx, target_table, context_table)
    return out.reshape(_BATCH)
